# baseline (speedup 1.0000x reference)
import functools
import numpy as np
import jax
import jax.numpy as jnp
from jax import lax
from jax.experimental import pallas as pl
from jax.experimental.pallas import tpu as pltpu
from jax.experimental.pallas import tpu_sc as plsc

NUM_FIELDS = 26
EMBED_DIM = 16
BATCH = 4096
NUM_PAIRS = (NUM_FIELDS * (NUM_FIELDS - 1)) // 2
FEAT = NUM_FIELDS * EMBED_DIM
HIDDEN = 400
TABLE_ROWS = 2600000

_OFFSETS = np.arange(NUM_FIELDS, dtype=np.int32) * 100000

_pairs_orig = [(i, j) for i in range(NUM_FIELDS) for j in range(i + 1, NUM_FIELDS)]
_pairs_new = [(i, i + k) for k in range(1, NUM_FIELDS) for i in range(NUM_FIELDS - k)]
_orig_pos = {p: n for n, p in enumerate(_pairs_orig)}
_PERM = np.array([_orig_pos[p] for p in _pairs_new], dtype=np.int32)

_NC = 2
_NS = 16
_NW = _NC * _NS
_SAMPLES_PER_W = BATCH // _NW
_ROWS_PER_W = _SAMPLES_PER_W * NUM_FIELDS
_HALF = _SAMPLES_PER_W // 2

_BB = 512
_NBLK = BATCH // _BB


_SUPER_COLS = 128
_TBLK_ROWS = TABLE_ROWS * EMBED_DIM // _SUPER_COLS
_CHUNK_ROWS = 128
_NCHUNK = _ROWS_PER_W // _CHUNK_ROWS
_NGRP = _CHUNK_ROWS // EMBED_DIM


def _sc_gather_t(tblk, super2, rem16, f16m, colm):
    mesh = plsc.VectorSubcoreMesh(core_axis_name="c", subcore_axis_name="s")

    @functools.partial(
        pl.kernel,
        mesh=mesh,
        compiler_params=pltpu.CompilerParams(
            use_tc_tiling_on_sc=True, needs_layout_passes=False
        ),
        out_type=jax.ShapeDtypeStruct((FEAT, BATCH), jnp.float32),
        scratch_types=[
            pltpu.VMEM((_NCHUNK, _CHUNK_ROWS), jnp.int32),
            pltpu.VMEM((_NCHUNK, _CHUNK_ROWS), jnp.int32),
            pltpu.VMEM((_NCHUNK, _CHUNK_ROWS), jnp.int32),
            pltpu.VMEM((_NCHUNK, _CHUNK_ROWS), jnp.int32),
            pltpu.VMEM((_CHUNK_ROWS, _SUPER_COLS), jnp.float32),
            pltpu.VMEM((_CHUNK_ROWS, _SUPER_COLS), jnp.float32),
            pltpu.VMEM((FEAT, _SAMPLES_PER_W), jnp.float32),
            pltpu.SemaphoreType.DMA,
            pltpu.SemaphoreType.DMA,
        ],
    )
    def k(tblk_hbm, super_hbm, rem_hbm, f16_hbm, col_hbm, out_hbm,
          super_v, rem_v, f16_v, col_v, chunk_a, chunk_b, buf, sem_a, sem_b):
        wid = lax.axis_index("s") * _NC + lax.axis_index("c")
        pltpu.sync_copy(super_hbm.at[wid], super_v)
        pltpu.sync_copy(rem_hbm.at[wid], rem_v)
        pltpu.sync_copy(f16_hbm, f16_v)
        pltpu.sync_copy(col_hbm, col_v)
        lane = jax.lax.broadcasted_iota(jnp.int32, (EMBED_DIM,), 0)

        def start(j, buf_ref, sem):
            return pltpu.async_copy(tblk_hbm.at[super_v.at[j]], buf_ref, sem)

        def extract(j, chunk):
            for g in range(_NGRP):
                sl = pl.ds(g * EMBED_DIM, EMBED_DIM)
                rem_vec = rem_v[j, sl]
                f16_vec = f16_v[j, sl]
                col_vec = col_v[j, sl]
                rows = g * EMBED_DIM + lane
                for d in range(EMBED_DIM):
                    val = plsc.load_gather(chunk, [rows, rem_vec + d])
                    plsc.store_scatter(buf, [f16_vec + d, col_vec], val)

        start(0, chunk_a, sem_a)
        start(1, chunk_b, sem_b)

        def ring(jj, carry):
            j_a = jj * 2
            j_b = j_a + 1
            pltpu.make_async_copy(tblk_hbm.at[super_v.at[j_a]], chunk_a, sem_a).wait()
            extract(j_a, chunk_a)

            @pl.when(j_a + 2 < _NCHUNK)
            def _():
                start(j_a + 2, chunk_a, sem_a)

            pltpu.make_async_copy(tblk_hbm.at[super_v.at[j_b]], chunk_b, sem_b).wait()
            extract(j_b, chunk_b)

            @pl.when(j_b + 2 < _NCHUNK)
            def _():
                start(j_b + 2, chunk_b, sem_b)

            return carry

        lax.fori_loop(0, _NCHUNK // 2, ring, 0)
        pltpu.sync_copy(buf, out_hbm.at[:, pl.ds(wid * _SAMPLES_PER_W, _SAMPLES_PER_W)])

    return k(tblk, super2, rem16, f16m, colm)


def _tc_body(e_ref, w0a_ref, w0b_ref, b0_ref, w1_ref, b1_ref, w2_ref, b2_ref,
             wo_ref, bo_ref, o_ref):
    et = e_ref[...]
    et3 = et.reshape(NUM_FIELDS, EMBED_DIM, _BB)
    parts = []
    for k in range(1, NUM_FIELDS):
        prod = et3[: NUM_FIELDS - k] * et3[k:]
        parts.append(jnp.sum(prod, axis=1))
    inner_t = jnp.concatenate(parts, axis=0)
    h = jnp.dot(w0a_ref[...], et, preferred_element_type=jnp.float32)
    h = h + jnp.dot(w0b_ref[...], inner_t, preferred_element_type=jnp.float32)
    h = jnp.maximum(h + b0_ref[...], 0.0)
    h = jnp.dot(w1_ref[...], h, preferred_element_type=jnp.float32) + b1_ref[...]
    h = jnp.maximum(h, 0.0)
    h = jnp.dot(w2_ref[...], h, preferred_element_type=jnp.float32) + b2_ref[...]
    h = jnp.maximum(h, 0.0)
    o_ref[...] = jnp.dot(wo_ref[...], h, preferred_element_type=jnp.float32) + bo_ref[...]


def _tc_forward(et, w0a_t, w0b_t, b0, w1_t, b1, w2_t, b2, wo_t, bo):
    full = lambda shape: pl.BlockSpec(shape, lambda i: (0,) * len(shape))
    return pl.pallas_call(
        _tc_body,
        grid=(_NBLK,),
        in_specs=[
            pl.BlockSpec((FEAT, _BB), lambda i: (0, i)),
            full((HIDDEN, FEAT)),
            full((HIDDEN, NUM_PAIRS)),
            full((HIDDEN, 1)),
            full((HIDDEN, HIDDEN)),
            full((HIDDEN, 1)),
            full((HIDDEN, HIDDEN)),
            full((HIDDEN, 1)),
            full((1, HIDDEN)),
            full((1, 1)),
        ],
        out_specs=pl.BlockSpec((1, _BB), lambda i: (0, i)),
        out_shape=jax.ShapeDtypeStruct((1, BATCH), jnp.float32),
    )(et, w0a_t, w0b_t, b0, w1_t, b1, w2_t, b2, wo_t, bo)


def kernel(x, params):
    tblk = params["table"].reshape(_TBLK_ROWS, _SUPER_COLS)
    idx = (x + _OFFSETS[None, :]).reshape(-1)
    super2 = (idx >> 3).reshape(_NW, _NCHUNK, _CHUNK_ROWS)
    rem16 = ((idx & 7) << 4).reshape(_NW, _NCHUNK, _CHUNK_ROWS)
    lrow = np.arange(_ROWS_PER_W, dtype=np.int32)
    f16m = jnp.asarray(((lrow % NUM_FIELDS) * EMBED_DIM).reshape(_NCHUNK, _CHUNK_ROWS))
    colm = jnp.asarray((lrow // NUM_FIELDS).reshape(_NCHUNK, _CHUNK_ROWS))
    et = _sc_gather_t(tblk, super2, rem16, f16m, colm)

    s = 1.0 / jnp.sqrt(1.0 + 1e-5)
    scale0 = params["g0"] * s
    scale1 = params["g1"] * s
    scale2 = params["g2"] * s
    w0 = params["W0"] * scale0[None, :]
    b0 = params["b0"] * scale0 + params["beta0"]
    w1 = params["W1"] * scale1[None, :]
    b1 = params["b1"] * scale1 + params["beta1"]
    w2 = params["W2"] * scale2[None, :]
    b2 = params["b2"] * scale2 + params["beta2"]
    w0a_t = w0[:FEAT].T
    w0b_t = w0[FEAT:][_PERM].T
    out_t = _tc_forward(
        et,
        w0a_t,
        w0b_t,
        b0[:, None],
        w1.T,
        b1[:, None],
        w2.T,
        b2[:, None],
        params["Wo"].T,
        params["bo"][:, None],
    )
    return out_t.reshape(BATCH, 1)


# speedup vs baseline: 1.1196x; 1.1196x over previous
import functools
import numpy as np
import jax
import jax.numpy as jnp
from jax import lax
from jax.experimental import pallas as pl
from jax.experimental.pallas import tpu as pltpu
from jax.experimental.pallas import tpu_sc as plsc

NUM_FIELDS = 26
EMBED_DIM = 16
BATCH = 4096
NUM_PAIRS = (NUM_FIELDS * (NUM_FIELDS - 1)) // 2
FEAT = NUM_FIELDS * EMBED_DIM
HIDDEN = 400
TABLE_ROWS = 2600000

_OFFSETS = np.arange(NUM_FIELDS, dtype=np.int32) * 100000

_pairs_orig = [(i, j) for i in range(NUM_FIELDS) for j in range(i + 1, NUM_FIELDS)]
_pairs_new = [(i, i + k) for k in range(1, NUM_FIELDS) for i in range(NUM_FIELDS - k)]
_orig_pos = {p: n for n, p in enumerate(_pairs_orig)}
_PERM = np.array([_orig_pos[p] for p in _pairs_new], dtype=np.int32)

_NC = 2
_NS = 16
_NW = _NC * _NS
_SAMPLES_PER_W = BATCH // _NW
_ROWS_PER_W = _SAMPLES_PER_W * NUM_FIELDS

_SUPER_COLS = 128
_TBLK_ROWS = TABLE_ROWS * EMBED_DIM // _SUPER_COLS
_CHUNK_ROWS = 128
_NCHUNK = _ROWS_PER_W // _CHUNK_ROWS
_NGRP = _CHUNK_ROWS // EMBED_DIM

_NTILE_FULL = TABLE_ROWS // _SUPER_COLS
_TAIL_COLS = TABLE_ROWS - _NTILE_FULL * _SUPER_COLS
_TILES_PER_W = -(-_NTILE_FULL // _NW)

_BB = 512
_NBLK = BATCH // _BB

_SC_PARAMS = pltpu.CompilerParams(use_tc_tiling_on_sc=True, needs_layout_passes=False)


def _sc_relayout(table_t, tail2):
    mesh = plsc.VectorSubcoreMesh(core_axis_name="c", subcore_axis_name="s")

    @functools.partial(
        pl.kernel,
        mesh=mesh,
        compiler_params=_SC_PARAMS,
        out_type=jax.ShapeDtypeStruct((_TBLK_ROWS, _SUPER_COLS), jnp.float32),
        scratch_types=[
            pltpu.VMEM((EMBED_DIM, _SUPER_COLS), jnp.float32),
            pltpu.VMEM((EMBED_DIM, _SUPER_COLS), jnp.float32),
            pltpu.VMEM((EMBED_DIM, _SUPER_COLS), jnp.float32),
            pltpu.VMEM((EMBED_DIM, _SUPER_COLS), jnp.float32),
            pltpu.SemaphoreType.DMA,
            pltpu.SemaphoreType.DMA,
            pltpu.SemaphoreType.DMA,
            pltpu.SemaphoreType.DMA,
        ],
    )
    def k(tt_hbm, tail_hbm, out_hbm, in_a, in_b, out_a, out_b, si_a, si_b, so_a, so_b):
        wid = lax.axis_index("s") * _NC + lax.axis_index("c")
        base = wid * _TILES_PER_W
        dvec = jax.lax.broadcasted_iota(jnp.int32, (EMBED_DIM,), 0)

        def src(c):
            return tt_hbm.at[:, pl.ds(pl.multiple_of(c * _SUPER_COLS, _SUPER_COLS),
                                      _SUPER_COLS)]

        def dst(c):
            return out_hbm.at[pl.ds(pl.multiple_of(c * EMBED_DIM, 8), EMBED_DIM), :]

        def transpose_tile(in_ref, out_ref):
            for t in range(EMBED_DIM):
                for kk in range(8):
                    col = jnp.full((EMBED_DIM,), 8 * t + kk, jnp.int32)
                    val = plsc.load_gather(in_ref, [dvec, col])
                    out_ref[t, pl.ds(kk * EMBED_DIM, EMBED_DIM)] = val

        pltpu.async_copy(src(base), in_a, si_a)
        pltpu.async_copy(src(base + 1), in_b, si_b)

        def step(i2, carry):
            for off, in_x, out_x, si_x, so_x in (
                (0, in_a, out_a, si_a, so_a),
                (1, in_b, out_b, si_b, so_b),
            ):
                j = i2 * 2 + off
                c = base + j

                @pl.when(jnp.logical_and(j < _TILES_PER_W, c < _NTILE_FULL))
                def _():
                    pltpu.make_async_copy(src(c), in_x, si_x).wait()

                    @pl.when(j >= 2)
                    def _():
                        pltpu.make_async_copy(out_x, dst(c), so_x).wait()

                    transpose_tile(in_x, out_x)
                    pltpu.async_copy(out_x, dst(c), so_x)
                    nc = c + 2

                    @pl.when(jnp.logical_and(j + 2 < _TILES_PER_W, nc < _NTILE_FULL))
                    def _():
                        pltpu.async_copy(src(nc), in_x, si_x)

            return carry

        lax.fori_loop(0, (_TILES_PER_W + 1) // 2, step, 0)
        last_a = base + ((_TILES_PER_W - 1) & ~1)
        last_b = base + 1 + (((_TILES_PER_W - 2) >> 1) << 1)
        pltpu.make_async_copy(out_a, dst(jnp.minimum(last_a, _NTILE_FULL - 1)),
                              so_a).wait()
        pltpu.make_async_copy(out_b, dst(jnp.minimum(last_b, _NTILE_FULL - 1)),
                              so_b).wait()

        @pl.when(wid == _NW - 1)
        def _():
            pltpu.sync_copy(tail_hbm, in_a.at[pl.ds(0, 8), :])
            pltpu.sync_copy(
                in_a.at[pl.ds(0, 8), :],
                out_hbm.at[pl.ds(_NTILE_FULL * EMBED_DIM, _TAIL_COLS // 8), :],
            )

    return k(table_t, tail2)


def _sc_gather_t(tblk, super2, rem16, f16m, colm):
    mesh = plsc.VectorSubcoreMesh(core_axis_name="c", subcore_axis_name="s")

    @functools.partial(
        pl.kernel,
        mesh=mesh,
        compiler_params=_SC_PARAMS,
        out_type=jax.ShapeDtypeStruct((FEAT, BATCH), jnp.float32),
        scratch_types=[
            pltpu.VMEM((_NCHUNK, _CHUNK_ROWS), jnp.int32),
            pltpu.VMEM((_NCHUNK, _CHUNK_ROWS), jnp.int32),
            pltpu.VMEM((_NCHUNK, _CHUNK_ROWS), jnp.int32),
            pltpu.VMEM((_NCHUNK, _CHUNK_ROWS), jnp.int32),
            pltpu.VMEM((_CHUNK_ROWS, _SUPER_COLS), jnp.float32),
            pltpu.VMEM((_CHUNK_ROWS, _SUPER_COLS), jnp.float32),
            pltpu.VMEM((FEAT, _SAMPLES_PER_W), jnp.float32),
            pltpu.SemaphoreType.DMA,
            pltpu.SemaphoreType.DMA,
        ],
    )
    def k(tblk_hbm, super_hbm, rem_hbm, f16_hbm, col_hbm, out_hbm,
          super_v, rem_v, f16_v, col_v, chunk_a, chunk_b, buf, sem_a, sem_b):
        wid = lax.axis_index("s") * _NC + lax.axis_index("c")
        pltpu.sync_copy(super_hbm.at[wid], super_v)
        pltpu.sync_copy(rem_hbm.at[wid], rem_v)
        pltpu.sync_copy(f16_hbm, f16_v)
        pltpu.sync_copy(col_hbm, col_v)
        lane = jax.lax.broadcasted_iota(jnp.int32, (EMBED_DIM,), 0)

        def start(j, buf_ref, sem):
            return pltpu.async_copy(tblk_hbm.at[super_v.at[j]], buf_ref, sem)

        def extract(j, chunk):
            for g in range(_NGRP):
                sl = pl.ds(g * EMBED_DIM, EMBED_DIM)
                rem_vec = rem_v[j, sl]
                f16_vec = f16_v[j, sl]
                col_vec = col_v[j, sl]
                rows = g * EMBED_DIM + lane
                for d in range(EMBED_DIM):
                    val = plsc.load_gather(chunk, [rows, rem_vec + d])
                    plsc.store_scatter(buf, [f16_vec + d, col_vec], val)

        start(0, chunk_a, sem_a)
        start(1, chunk_b, sem_b)

        def ring(jj, carry):
            j_a = jj * 2
            j_b = j_a + 1
            pltpu.make_async_copy(tblk_hbm.at[super_v.at[j_a]], chunk_a, sem_a).wait()
            extract(j_a, chunk_a)

            @pl.when(j_a + 2 < _NCHUNK)
            def _():
                start(j_a + 2, chunk_a, sem_a)

            pltpu.make_async_copy(tblk_hbm.at[super_v.at[j_b]], chunk_b, sem_b).wait()
            extract(j_b, chunk_b)

            @pl.when(j_b + 2 < _NCHUNK)
            def _():
                start(j_b + 2, chunk_b, sem_b)

            return carry

        lax.fori_loop(0, _NCHUNK // 2, ring, 0)
        pltpu.sync_copy(buf, out_hbm.at[:, pl.ds(wid * _SAMPLES_PER_W, _SAMPLES_PER_W)])

    return k(tblk, super2, rem16, f16m, colm)


def _tc_body(e_ref, w0a_ref, w0b_ref, b0_ref, w1_ref, b1_ref, w2_ref, b2_ref,
             wo_ref, bo_ref, o_ref):
    et = e_ref[...]
    et3 = et.reshape(NUM_FIELDS, EMBED_DIM, _BB)
    parts = []
    for k in range(1, NUM_FIELDS):
        prod = et3[: NUM_FIELDS - k] * et3[k:]
        parts.append(jnp.sum(prod, axis=1))
    inner_t = jnp.concatenate(parts, axis=0)
    h = jnp.dot(w0a_ref[...], et, preferred_element_type=jnp.float32)
    h = h + jnp.dot(w0b_ref[...], inner_t, preferred_element_type=jnp.float32)
    h = jnp.maximum(h + b0_ref[...], 0.0)
    h = jnp.dot(w1_ref[...], h, preferred_element_type=jnp.float32) + b1_ref[...]
    h = jnp.maximum(h, 0.0)
    h = jnp.dot(w2_ref[...], h, preferred_element_type=jnp.float32) + b2_ref[...]
    h = jnp.maximum(h, 0.0)
    o_ref[...] = jnp.dot(wo_ref[...], h, preferred_element_type=jnp.float32) + bo_ref[...]


def _tc_forward(et, w0a_t, w0b_t, b0, w1_t, b1, w2_t, b2, wo_t, bo):
    full = lambda shape: pl.BlockSpec(shape, lambda i: (0,) * len(shape))
    return pl.pallas_call(
        _tc_body,
        grid=(_NBLK,),
        in_specs=[
            pl.BlockSpec((FEAT, _BB), lambda i: (0, i)),
            full((HIDDEN, FEAT)),
            full((HIDDEN, NUM_PAIRS)),
            full((HIDDEN, 1)),
            full((HIDDEN, HIDDEN)),
            full((HIDDEN, 1)),
            full((HIDDEN, HIDDEN)),
            full((HIDDEN, 1)),
            full((1, HIDDEN)),
            full((1, 1)),
        ],
        out_specs=pl.BlockSpec((1, _BB), lambda i: (0, i)),
        out_shape=jax.ShapeDtypeStruct((1, BATCH), jnp.float32),
    )(et, w0a_t, w0b_t, b0, w1_t, b1, w2_t, b2, wo_t, bo)


def kernel(x, params):
    table_t = jnp.transpose(params["table"])
    tail2 = params["table"][_NTILE_FULL * _SUPER_COLS:].reshape(8, _SUPER_COLS)
    tblk = _sc_relayout(table_t, tail2)
    idx = (x + _OFFSETS[None, :]).reshape(-1)
    super2 = (idx >> 3).reshape(_NW, _NCHUNK, _CHUNK_ROWS)
    rem16 = ((idx & 7) << 4).reshape(_NW, _NCHUNK, _CHUNK_ROWS)
    lrow = np.arange(_ROWS_PER_W, dtype=np.int32)
    f16m = jnp.asarray(((lrow % NUM_FIELDS) * EMBED_DIM).reshape(_NCHUNK, _CHUNK_ROWS))
    colm = jnp.asarray((lrow // NUM_FIELDS).reshape(_NCHUNK, _CHUNK_ROWS))
    et = _sc_gather_t(tblk, super2, rem16, f16m, colm)

    s = 1.0 / jnp.sqrt(1.0 + 1e-5)
    scale0 = params["g0"] * s
    scale1 = params["g1"] * s
    scale2 = params["g2"] * s
    w0 = params["W0"] * scale0[None, :]
    b0 = params["b0"] * scale0 + params["beta0"]
    w1 = params["W1"] * scale1[None, :]
    b1 = params["b1"] * scale1 + params["beta1"]
    w2 = params["W2"] * scale2[None, :]
    b2 = params["b2"] * scale2 + params["beta2"]
    w0a_t = w0[:FEAT].T
    w0b_t = w0[FEAT:][_PERM].T
    out_t = _tc_forward(
        et,
        w0a_t,
        w0b_t,
        b0[:, None],
        w1.T,
        b1[:, None],
        w2.T,
        b2[:, None],
        params["Wo"].T,
        params["bo"][:, None],
    )
    return out_t.reshape(BATCH, 1)


# speedup vs baseline: 1.5353x; 1.3712x over previous
import functools
import numpy as np
import jax
import jax.numpy as jnp
from jax import lax
from jax.experimental import pallas as pl
from jax.experimental.pallas import tpu as pltpu
from jax.experimental.pallas import tpu_sc as plsc

NUM_FIELDS = 26
EMBED_DIM = 16
BATCH = 4096
NUM_PAIRS = (NUM_FIELDS * (NUM_FIELDS - 1)) // 2
FEAT = NUM_FIELDS * EMBED_DIM
HIDDEN = 400
TABLE_ROWS = 2600000

_OFFSETS = np.arange(NUM_FIELDS, dtype=np.int32) * 100000

_pairs_orig = [(i, j) for i in range(NUM_FIELDS) for j in range(i + 1, NUM_FIELDS)]
_pairs_new = [(i, i + k) for k in range(1, NUM_FIELDS) for i in range(NUM_FIELDS - k)]
_orig_pos = {p: n for n, p in enumerate(_pairs_orig)}
_PERM = np.array([_orig_pos[p] for p in _pairs_new], dtype=np.int32)

_NC = 2
_NS = 16
_NW = _NC * _NS
_SAMPLES_PER_W = BATCH // _NW
_ROWS_PER_W = _SAMPLES_PER_W * NUM_FIELDS

_SUPER_COLS = 128
_TBLK_ROWS = TABLE_ROWS * EMBED_DIM // _SUPER_COLS
_CHUNK_ROWS = 128
_NCHUNK = _ROWS_PER_W // _CHUNK_ROWS
_NGRP = _CHUNK_ROWS // EMBED_DIM

_NTILE_FULL = TABLE_ROWS // _SUPER_COLS
_TAIL_COLS = TABLE_ROWS - _NTILE_FULL * _SUPER_COLS
_TILES_PER_W = -(-_NTILE_FULL // _NW)

_BB = 512
_NBLK = BATCH // _BB

_SC_PARAMS = pltpu.CompilerParams(use_tc_tiling_on_sc=True, needs_layout_passes=False)


def _sc_relayout(table_t, tail2):
    mesh = plsc.VectorSubcoreMesh(core_axis_name="c", subcore_axis_name="s")

    @functools.partial(
        pl.kernel,
        mesh=mesh,
        compiler_params=_SC_PARAMS,
        out_type=jax.ShapeDtypeStruct((_TBLK_ROWS, _SUPER_COLS), jnp.float32),
        scratch_types=[
            pltpu.VMEM((EMBED_DIM, _SUPER_COLS), jnp.float32),
            pltpu.VMEM((EMBED_DIM, _SUPER_COLS), jnp.float32),
            pltpu.VMEM((EMBED_DIM, _SUPER_COLS), jnp.float32),
            pltpu.VMEM((EMBED_DIM, _SUPER_COLS), jnp.float32),
            pltpu.VMEM((EMBED_DIM * _SUPER_COLS,), jnp.float32),
            pltpu.SemaphoreType.DMA,
            pltpu.SemaphoreType.DMA,
            pltpu.SemaphoreType.DMA,
            pltpu.SemaphoreType.DMA,
        ],
    )
    def k(tt_hbm, tail_hbm, out_hbm, in_a, in_b, out_a, out_b, skew,
          si_a, si_b, so_a, so_b):
        wid = lax.axis_index("s") * _NC + lax.axis_index("c")
        base = wid * _TILES_PER_W
        lane = jax.lax.broadcasted_iota(jnp.int32, (EMBED_DIM,), 0)
        perm = [(lane + m) & 15 for m in range(EMBED_DIM)]
        lane16 = lane * EMBED_DIM

        def src(c):
            return tt_hbm.at[:, pl.ds(pl.multiple_of(c * _SUPER_COLS, _SUPER_COLS),
                                      _SUPER_COLS)]

        def dst(c):
            return out_hbm.at[pl.ds(pl.multiple_of(c * EMBED_DIM, 8), EMBED_DIM), :]

        def transpose_tile(in_ref, out_ref):
            for d in range(EMBED_DIM):
                aidx = lane16 + perm[d]
                for kk in range(8):
                    v = in_ref[d, pl.ds(kk * EMBED_DIM, EMBED_DIM)]
                    plsc.store_scatter(skew, [aidx + kk * 256], v)
            for t in range(EMBED_DIM):
                for kk in range(8):
                    c = 8 * t + kk
                    bidx = perm[c & 15] + c * EMBED_DIM
                    val = plsc.load_gather(skew, [bidx])
                    out_ref[t, pl.ds(kk * EMBED_DIM, EMBED_DIM)] = val

        pltpu.async_copy(src(base), in_a, si_a)
        pltpu.async_copy(src(base + 1), in_b, si_b)

        def step(i2, carry):
            for off, in_x, out_x, si_x, so_x in (
                (0, in_a, out_a, si_a, so_a),
                (1, in_b, out_b, si_b, so_b),
            ):
                j = i2 * 2 + off
                c = base + j

                @pl.when(jnp.logical_and(j < _TILES_PER_W, c < _NTILE_FULL))
                def _():
                    pltpu.make_async_copy(src(c), in_x, si_x).wait()

                    @pl.when(j >= 2)
                    def _():
                        pltpu.make_async_copy(out_x, dst(c), so_x).wait()

                    transpose_tile(in_x, out_x)
                    pltpu.async_copy(out_x, dst(c), so_x)
                    nc = c + 2

                    @pl.when(jnp.logical_and(j + 2 < _TILES_PER_W, nc < _NTILE_FULL))
                    def _():
                        pltpu.async_copy(src(nc), in_x, si_x)

            return carry

        lax.fori_loop(0, (_TILES_PER_W + 1) // 2, step, 0)
        last_a = base + ((_TILES_PER_W - 1) & ~1)
        last_b = base + 1 + (((_TILES_PER_W - 2) >> 1) << 1)
        pltpu.make_async_copy(out_a, dst(jnp.minimum(last_a, _NTILE_FULL - 1)),
                              so_a).wait()
        pltpu.make_async_copy(out_b, dst(jnp.minimum(last_b, _NTILE_FULL - 1)),
                              so_b).wait()

        @pl.when(wid == _NW - 1)
        def _():
            pltpu.sync_copy(tail_hbm, in_a.at[pl.ds(0, 8), :])
            pltpu.sync_copy(
                in_a.at[pl.ds(0, 8), :],
                out_hbm.at[pl.ds(_NTILE_FULL * EMBED_DIM, _TAIL_COLS // 8), :],
            )

    return k(table_t, tail2)


def _sc_gather_t(tblk, super2, rem16, f16m, colm):
    mesh = plsc.VectorSubcoreMesh(core_axis_name="c", subcore_axis_name="s")

    @functools.partial(
        pl.kernel,
        mesh=mesh,
        compiler_params=_SC_PARAMS,
        out_type=jax.ShapeDtypeStruct((FEAT, BATCH), jnp.float32),
        scratch_types=[
            pltpu.VMEM((_NCHUNK, _CHUNK_ROWS), jnp.int32),
            pltpu.VMEM((_NCHUNK, _CHUNK_ROWS), jnp.int32),
            pltpu.VMEM((_NCHUNK, _CHUNK_ROWS), jnp.int32),
            pltpu.VMEM((_NCHUNK, _CHUNK_ROWS), jnp.int32),
            pltpu.VMEM((_CHUNK_ROWS, _SUPER_COLS), jnp.float32),
            pltpu.VMEM((_CHUNK_ROWS, _SUPER_COLS), jnp.float32),
            pltpu.VMEM((FEAT, _SAMPLES_PER_W), jnp.float32),
            pltpu.SemaphoreType.DMA,
            pltpu.SemaphoreType.DMA,
        ],
    )
    def k(tblk_hbm, super_hbm, rem_hbm, f16_hbm, col_hbm, out_hbm,
          super_v, rem_v, f16_v, col_v, chunk_a, chunk_b, buf, sem_a, sem_b):
        wid = lax.axis_index("s") * _NC + lax.axis_index("c")
        pltpu.sync_copy(super_hbm.at[wid], super_v)
        pltpu.sync_copy(rem_hbm.at[wid], rem_v)
        pltpu.sync_copy(f16_hbm, f16_v)
        pltpu.sync_copy(col_hbm, col_v)
        lane = jax.lax.broadcasted_iota(jnp.int32, (EMBED_DIM,), 0)

        def start(j, buf_ref, sem):
            return pltpu.async_copy(tblk_hbm.at[super_v.at[j]], buf_ref, sem)

        def extract(j, chunk):
            for g in range(_NGRP):
                sl = pl.ds(g * EMBED_DIM, EMBED_DIM)
                rem_vec = rem_v[j, sl]
                f16_vec = f16_v[j, sl]
                col_vec = col_v[j, sl]
                rows = g * EMBED_DIM + lane
                for d in range(EMBED_DIM):
                    val = plsc.load_gather(chunk, [rows, rem_vec + d])
                    plsc.store_scatter(buf, [f16_vec + d, col_vec], val)

        start(0, chunk_a, sem_a)
        start(1, chunk_b, sem_b)

        def ring(jj, carry):
            j_a = jj * 2
            j_b = j_a + 1
            pltpu.make_async_copy(tblk_hbm.at[super_v.at[j_a]], chunk_a, sem_a).wait()
            extract(j_a, chunk_a)

            @pl.when(j_a + 2 < _NCHUNK)
            def _():
                start(j_a + 2, chunk_a, sem_a)

            pltpu.make_async_copy(tblk_hbm.at[super_v.at[j_b]], chunk_b, sem_b).wait()
            extract(j_b, chunk_b)

            @pl.when(j_b + 2 < _NCHUNK)
            def _():
                start(j_b + 2, chunk_b, sem_b)

            return carry

        lax.fori_loop(0, _NCHUNK // 2, ring, 0)
        pltpu.sync_copy(buf, out_hbm.at[:, pl.ds(wid * _SAMPLES_PER_W, _SAMPLES_PER_W)])

    return k(tblk, super2, rem16, f16m, colm)


def _tc_body(e_ref, w0a_ref, w0b_ref, b0_ref, w1_ref, b1_ref, w2_ref, b2_ref,
             wo_ref, bo_ref, o_ref):
    et = e_ref[...]
    et3 = et.reshape(NUM_FIELDS, EMBED_DIM, _BB)
    parts = []
    for k in range(1, NUM_FIELDS):
        prod = et3[: NUM_FIELDS - k] * et3[k:]
        parts.append(jnp.sum(prod, axis=1))
    inner_t = jnp.concatenate(parts, axis=0)
    h = jnp.dot(w0a_ref[...], et, preferred_element_type=jnp.float32)
    h = h + jnp.dot(w0b_ref[...], inner_t, preferred_element_type=jnp.float32)
    h = jnp.maximum(h + b0_ref[...], 0.0)
    h = jnp.dot(w1_ref[...], h, preferred_element_type=jnp.float32) + b1_ref[...]
    h = jnp.maximum(h, 0.0)
    h = jnp.dot(w2_ref[...], h, preferred_element_type=jnp.float32) + b2_ref[...]
    h = jnp.maximum(h, 0.0)
    o_ref[...] = jnp.dot(wo_ref[...], h, preferred_element_type=jnp.float32) + bo_ref[...]


def _tc_forward(et, w0a_t, w0b_t, b0, w1_t, b1, w2_t, b2, wo_t, bo):
    full = lambda shape: pl.BlockSpec(shape, lambda i: (0,) * len(shape))
    return pl.pallas_call(
        _tc_body,
        grid=(_NBLK,),
        in_specs=[
            pl.BlockSpec((FEAT, _BB), lambda i: (0, i)),
            full((HIDDEN, FEAT)),
            full((HIDDEN, NUM_PAIRS)),
            full((HIDDEN, 1)),
            full((HIDDEN, HIDDEN)),
            full((HIDDEN, 1)),
            full((HIDDEN, HIDDEN)),
            full((HIDDEN, 1)),
            full((1, HIDDEN)),
            full((1, 1)),
        ],
        out_specs=pl.BlockSpec((1, _BB), lambda i: (0, i)),
        out_shape=jax.ShapeDtypeStruct((1, BATCH), jnp.float32),
    )(et, w0a_t, w0b_t, b0, w1_t, b1, w2_t, b2, wo_t, bo)


def kernel(x, params):
    table_t = jnp.transpose(params["table"])
    tail2 = params["table"][_NTILE_FULL * _SUPER_COLS:].reshape(8, _SUPER_COLS)
    tblk = _sc_relayout(table_t, tail2)
    idx = (x + _OFFSETS[None, :]).reshape(-1)
    super2 = (idx >> 3).reshape(_NW, _NCHUNK, _CHUNK_ROWS)
    rem16 = ((idx & 7) << 4).reshape(_NW, _NCHUNK, _CHUNK_ROWS)
    lrow = np.arange(_ROWS_PER_W, dtype=np.int32)
    f16m = jnp.asarray(((lrow % NUM_FIELDS) * EMBED_DIM).reshape(_NCHUNK, _CHUNK_ROWS))
    colm = jnp.asarray((lrow // NUM_FIELDS).reshape(_NCHUNK, _CHUNK_ROWS))
    et = _sc_gather_t(tblk, super2, rem16, f16m, colm)

    s = 1.0 / jnp.sqrt(1.0 + 1e-5)
    scale0 = params["g0"] * s
    scale1 = params["g1"] * s
    scale2 = params["g2"] * s
    w0 = params["W0"] * scale0[None, :]
    b0 = params["b0"] * scale0 + params["beta0"]
    w1 = params["W1"] * scale1[None, :]
    b1 = params["b1"] * scale1 + params["beta1"]
    w2 = params["W2"] * scale2[None, :]
    b2 = params["b2"] * scale2 + params["beta2"]
    w0a_t = w0[:FEAT].T
    w0b_t = w0[FEAT:][_PERM].T
    out_t = _tc_forward(
        et,
        w0a_t,
        w0b_t,
        b0[:, None],
        w1.T,
        b1[:, None],
        w2.T,
        b2[:, None],
        params["Wo"].T,
        params["bo"][:, None],
    )
    return out_t.reshape(BATCH, 1)
